# Initial kernel scaffold; baseline (speedup 1.0000x reference)
import jax
import jax.numpy as jnp
from jax.experimental import pallas as pl


def kernel(x1, edge_index1, batch1, x2, edge_index2, batch2, W_gcn, b_gcn, W_link, b_link):
    raise NotImplementedError("write your pallas kernel here")


# speedup vs baseline: 12.1321x; 12.1321x over previous
import jax
import jax.numpy as jnp
from jax import lax
from jax.experimental import pallas as pl
from jax.experimental.pallas import tpu as pltpu
from jax.experimental.pallas import tpu_sc as plsc

N_NODES = 10000
D = 128
G = 128
CLS = 7

NC = 2
NS = 16
L = 16

N_PAD = 10240
NPW = N_PAD // NS
RB = 128
E_PAD = 327680
EPW = E_PAD // NS
NCHUNK = EPW // RB
NB = NPW // RB
GP = G + 1
GPP = 144
GPW = G // NS


def _mm_body(x_ref, w_ref, o_ref):
    o_ref[...] = jnp.dot(x_ref[...], w_ref[...],
                         preferred_element_type=jnp.float32)


def _matmul(x, w):
    m = x.shape[0]
    bm = 1280
    return pl.pallas_call(
        _mm_body,
        grid=(m // bm,),
        in_specs=[pl.BlockSpec((bm, D), lambda i: (i, 0)),
                  pl.BlockSpec((D, D), lambda i: (0, 0))],
        out_specs=pl.BlockSpec((bm, D), lambda i: (i, 0)),
        out_shape=jax.ShapeDtypeStruct((m, D), jnp.float32),
    )(x, w)


def _lin_body(p_ref, w_ref, b_ref, o_ref):
    p = p_ref[...]
    o_ref[...] = (jnp.dot(p[0], w_ref[:D], preferred_element_type=jnp.float32)
                  + jnp.dot(p[1], w_ref[D:], preferred_element_type=jnp.float32)
                  + b_ref[...])


def _linear(pooled, w_link, b_link):
    return pl.pallas_call(
        _lin_body,
        out_shape=jax.ShapeDtypeStruct((G, CLS), jnp.float32),
    )(pooled, w_link, b_link)


def _rsqrt16(x):
    y = x
    for _ in range(16):
        y = 0.5 * (y + x / y)
    return 1.0 / y


def _splat(ref, i):
    return plsc.load_gather(ref, [jnp.full((L,), i, jnp.int32)])


def _sc_body(xw_hbm, srcg_hbm, dstl_hbm, batch_hbm, bias_hbm,
             y_hbm, out_hbm,
             sidx, didx, tmp_v, dinv_v, rowbuf, cntv_v, batch_v,
             bias_v, ones_v, ptmp,
             hist_sh, acc_sh, pooled_sh, cnt_sh, sem):
    c = lax.axis_index("c")
    s = lax.axis_index("s")
    row0g = c * N_PAD + s * NPW
    row0l = s * NPW
    g0 = s * GPW

    zeros16 = jnp.zeros((L,), jnp.float32)
    ones16 = jnp.ones((L,), jnp.float32)

    pltpu.sync_copy(batch_hbm.at[c, s], batch_v)
    pltpu.sync_copy(bias_hbm, bias_v)

    def ztmp(i, _):
        tmp_v[pl.ds(i * L, L)] = zeros16
        return 0
    lax.fori_loop(0, NPW // L, ztmp, 0)
    pltpu.sync_copy(tmp_v, hist_sh.at[pl.ds(row0l, NPW)])

    def zcnt(i, _):
        cntv_v[pl.ds(i * L, L)] = zeros16
        return 0
    lax.fori_loop(0, GPP // L, zcnt, 0)

    def zpt(r, _):
        for j in range(D // L):
            ptmp[r, pl.ds(j * L, L)] = zeros16
        return 0
    lax.fori_loop(0, GPW, zpt, 0)
    pltpu.sync_copy(ptmp, pooled_sh.at[pl.ds(g0, GPW)])

    @pl.when(s == 0)
    def _():
        pltpu.sync_copy(cntv_v, cnt_sh)
        pltpu.sync_copy(ptmp.at[0], pooled_sh.at[G])

    def fones(i, _):
        ones_v[pl.ds(i * L, L)] = ones16
        return 0
    lax.fori_loop(0, RB // L, fones, 0)
    plsc.subcore_barrier()

    def hist_chunk(k, _):
        pltpu.sync_copy(dstl_hbm.at[c, s, k], didx.at[0])
        pltpu.sync_copy(ones_v, hist_sh.at[didx.at[0]], add=True)
        return 0
    lax.fori_loop(0, NCHUNK, hist_chunk, 0)

    def cnt_chunk(k, _):
        pltpu.sync_copy(ones_v, cnt_sh.at[batch_v.at[k]], add=True)
        return 0
    lax.fori_loop(0, NB, cnt_chunk, 0)
    plsc.subcore_barrier()

    pltpu.sync_copy(hist_sh.at[pl.ds(row0l, NPW)], tmp_v)

    def mkdinv(i, _):
        deg = tmp_v[pl.ds(i * L, L)] + 1.0
        dinv_v[pl.ds(i * L, L)] = _rsqrt16(deg)
        return 0
    lax.fori_loop(0, NPW // L, mkdinv, 0)

    def prescale(k, _):
        pltpu.sync_copy(xw_hbm.at[pl.ds(row0g + k * RB, RB)], rowbuf)
        def prow(r, _):
            val = _splat(dinv_v, k * RB + r)
            for j in range(D // L):
                rowbuf[r, pl.ds(j * L, L)] = rowbuf[r, pl.ds(j * L, L)] * val
            return 0
        lax.fori_loop(0, RB, prow, 0)
        pltpu.sync_copy(rowbuf, y_hbm.at[pl.ds(row0g + k * RB, RB)])
        pltpu.sync_copy(rowbuf, acc_sh.at[pl.ds(row0l + k * RB, RB)])
        return 0
    lax.fori_loop(0, NB, prescale, 0)
    plsc.subcore_barrier()

    def edge_chunk(k, _):
        pltpu.sync_copy(srcg_hbm.at[c, s, k], sidx.at[0])
        pltpu.sync_copy(dstl_hbm.at[c, s, k], didx.at[0])
        pltpu.async_copy(y_hbm.at[sidx.at[0]], rowbuf, sem).wait()
        pltpu.sync_copy(rowbuf, acc_sh.at[didx.at[0]], add=True)
        return 0
    lax.fori_loop(0, NCHUNK, edge_chunk, 0)
    plsc.subcore_barrier()

    def pool_chunk(k, _):
        pltpu.sync_copy(acc_sh.at[pl.ds(row0l + k * RB, RB)], rowbuf)
        def prow(r, _):
            val = _splat(dinv_v, k * RB + r)
            for j in range(D // L):
                rowbuf[r, pl.ds(j * L, L)] = jnp.maximum(
                    rowbuf[r, pl.ds(j * L, L)] * val
                    + bias_v[pl.ds(j * L, L)], 0.0)
            return 0
        lax.fori_loop(0, RB, prow, 0)
        pltpu.sync_copy(rowbuf, pooled_sh.at[batch_v.at[k]], add=True)
        return 0
    lax.fori_loop(0, NB, pool_chunk, 0)
    plsc.subcore_barrier()

    pltpu.sync_copy(cnt_sh, cntv_v)
    pltpu.sync_copy(pooled_sh.at[pl.ds(g0, GPW)], ptmp)

    def divrow(r, _):
        cvec = _splat(cntv_v, g0 + r)
        recip = 1.0 / jnp.maximum(cvec, 1.0)
        for j in range(D // L):
            ptmp[r, pl.ds(j * L, L)] = ptmp[r, pl.ds(j * L, L)] * recip
        return 0
    lax.fori_loop(0, GPW, divrow, 0)

    pltpu.sync_copy(ptmp, out_hbm.at[c, pl.ds(g0, GPW)])


def _sc_call(xw, srcg, dstl, batchp, bias):
    mesh = plsc.VectorSubcoreMesh(core_axis_name="c", subcore_axis_name="s",
                                  num_cores=NC, num_subcores=NS)
    f = pl.kernel(
        _sc_body,
        out_type=(jax.ShapeDtypeStruct((NC * N_PAD, D), jnp.float32),
                  jax.ShapeDtypeStruct((NC, G, D), jnp.float32)),
        mesh=mesh,
        compiler_params=pltpu.CompilerParams(needs_layout_passes=False),
        scratch_types=[
            pltpu.VMEM((2, RB), jnp.int32),
            pltpu.VMEM((2, RB), jnp.int32),
            pltpu.VMEM((NPW,), jnp.float32),
            pltpu.VMEM((NPW,), jnp.float32),
            pltpu.VMEM((RB, D), jnp.float32),
            pltpu.VMEM((GPP,), jnp.float32),
            pltpu.VMEM((NB, RB), jnp.int32),
            pltpu.VMEM((D,), jnp.float32),
            pltpu.VMEM((RB,), jnp.float32),
            pltpu.VMEM((GPW, D), jnp.float32),
            pltpu.VMEM_SHARED((N_PAD,), jnp.float32),
            pltpu.VMEM_SHARED((N_PAD, D), jnp.float32),
            pltpu.VMEM_SHARED((GP, D), jnp.float32),
            pltpu.VMEM_SHARED((GPP,), jnp.float32),
            pltpu.SemaphoreType.DMA,
        ],
    )
    return f(xw, srcg, dstl, batchp, bias)


def kernel(x1, edge_index1, batch1, x2, edge_index2, batch2,
           W_gcn, b_gcn, W_link, b_link):
    f32 = jnp.float32
    i32 = jnp.int32
    e1 = edge_index1.astype(i32)
    e2 = edge_index2.astype(i32)

    xp = jnp.concatenate([
        jnp.pad(x1.astype(f32), ((0, N_PAD - N_NODES), (0, 0))),
        jnp.pad(x2.astype(f32), ((0, N_PAD - N_NODES), (0, 0))),
    ], axis=0)
    xw = _matmul(xp, W_gcn.astype(f32))

    pad_e = E_PAD - e1.shape[1]

    def prep(e, off):
        src = jnp.concatenate(
            [e[0] + i32(off), jnp.full((pad_e,), off + N_PAD - 1, i32)])
        dst = jnp.concatenate(
            [e[1], jnp.full((pad_e,), N_PAD - 1, i32)])
        return src.reshape(NS, NCHUNK, RB), dst.reshape(NS, NCHUNK, RB)

    s1, d1 = prep(e1, 0)
    s2, d2 = prep(e2, N_PAD)
    srcg = jnp.stack([s1, s2])
    dstl = jnp.stack([d1, d2])
    batchp = jnp.stack([
        jnp.pad(batch1.astype(i32), (0, N_PAD - N_NODES), constant_values=G),
        jnp.pad(batch2.astype(i32), (0, N_PAD - N_NODES), constant_values=G),
    ]).reshape(NC, NS, NB, RB)

    _y, pooled = _sc_call(xw, srcg, dstl, batchp, b_gcn.astype(f32))
    return _linear(pooled, W_link.astype(f32),
                   b_link.reshape(1, CLS).astype(f32))


# speedup vs baseline: 16.7036x; 1.3768x over previous
import jax
import jax.numpy as jnp
from jax import lax
from jax.experimental import pallas as pl
from jax.experimental.pallas import tpu as pltpu
from jax.experimental.pallas import tpu_sc as plsc

N_NODES = 10000
D = 128
G = 128
CLS = 7

NC = 2
NS = 16
L = 16

N_PAD = 10240
NPW = N_PAD // NS
RB = 128
E_PAD = 327680
EPW = E_PAD // NS
NCHUNK = EPW // RB
SG = 32
NBLK = NCHUNK // SG
NB = NPW // RB
GP = G + 1
GPP = 144
GPW = G // NS


def _mm_body(x_ref, w_ref, o_ref):
    o_ref[...] = jnp.dot(x_ref[...], w_ref[...],
                         preferred_element_type=jnp.float32)


def _matmul(x, w):
    m = x.shape[0]
    bm = 1280
    return pl.pallas_call(
        _mm_body,
        grid=(m // bm,),
        in_specs=[pl.BlockSpec((bm, D), lambda i: (i, 0)),
                  pl.BlockSpec((D, D), lambda i: (0, 0))],
        out_specs=pl.BlockSpec((bm, D), lambda i: (i, 0)),
        out_shape=jax.ShapeDtypeStruct((m, D), jnp.float32),
    )(x, w)


def _lin_body(p_ref, w_ref, b_ref, o_ref):
    p = p_ref[...]
    o_ref[...] = (jnp.dot(p[0], w_ref[:D], preferred_element_type=jnp.float32)
                  + jnp.dot(p[1], w_ref[D:], preferred_element_type=jnp.float32)
                  + b_ref[...])


def _linear(pooled, w_link, b_link):
    return pl.pallas_call(
        _lin_body,
        out_shape=jax.ShapeDtypeStruct((G, CLS), jnp.float32),
    )(pooled, w_link, b_link)


def _rsqrt16(x):
    y = x
    for _ in range(16):
        y = 0.5 * (y + x / y)
    return 1.0 / y


def _splat(ref, i):
    return plsc.load_gather(ref, [jnp.full((L,), i, jnp.int32)])


def _sc_body(xw_hbm, srcg_hbm, dstl_hbm, batch_hbm, bias_hbm,
             y_hbm, out_hbm,
             sidx, didx, tmp_v, dinv_v, ebuf, cntv_v, batch_v,
             bias_v, ones_v, ptmp,
             hist_sh, acc_sh, pooled_sh, cnt_sh, sem, sem2):
    c = lax.axis_index("c")
    s = lax.axis_index("s")
    row0g = c * N_PAD + s * NPW
    row0l = s * NPW
    g0 = s * GPW

    bufA = ebuf.at[pl.ds(0, RB)]
    bufB = ebuf.at[pl.ds(RB, RB)]

    zeros16 = jnp.zeros((L,), jnp.float32)
    ones16 = jnp.ones((L,), jnp.float32)

    pltpu.sync_copy(batch_hbm.at[c, s], batch_v)
    pltpu.sync_copy(bias_hbm, bias_v)

    def ztmp(i, _):
        tmp_v[pl.ds(i * L, L)] = zeros16
        return 0
    lax.fori_loop(0, NPW // L, ztmp, 0)
    pltpu.sync_copy(tmp_v, hist_sh.at[pl.ds(row0l, NPW)])

    def zcnt(i, _):
        cntv_v[pl.ds(i * L, L)] = zeros16
        return 0
    lax.fori_loop(0, GPP // L, zcnt, 0)

    def zpt(r, _):
        for j in range(D // L):
            ptmp[r, pl.ds(j * L, L)] = zeros16
        return 0
    lax.fori_loop(0, GPW, zpt, 0)
    pltpu.sync_copy(ptmp, pooled_sh.at[pl.ds(g0, GPW)])

    @pl.when(s == 0)
    def _():
        pltpu.sync_copy(cntv_v, cnt_sh)
        pltpu.sync_copy(ptmp.at[0], pooled_sh.at[G])

    def fones(i, _):
        ones_v[pl.ds(i * L, L)] = ones16
        return 0
    lax.fori_loop(0, RB // L, fones, 0)
    plsc.subcore_barrier()

    def hist_blk(blk, _):
        pltpu.sync_copy(dstl_hbm.at[c, s, pl.ds(blk * SG, SG)], didx)

        def grp(j, _):
            cps = [pltpu.async_copy(ones_v, hist_sh.at[didx.at[j * 8 + b]],
                                    sem, add=True) for b in range(8)]
            for cp in cps:
                cp.wait()
            return 0
        lax.fori_loop(0, SG // 8, grp, 0)
        return 0
    lax.fori_loop(0, NBLK, hist_blk, 0)

    def cnt_chunk(k, _):
        pltpu.sync_copy(ones_v, cnt_sh.at[batch_v.at[k]], add=True)
        return 0
    lax.fori_loop(0, NB, cnt_chunk, 0)
    plsc.subcore_barrier()

    pltpu.sync_copy(hist_sh.at[pl.ds(row0l, NPW)], tmp_v)

    def mkdinv(i, _):
        deg = tmp_v[pl.ds(i * L, L)] + 1.0
        dinv_v[pl.ds(i * L, L)] = _rsqrt16(deg)
        return 0
    lax.fori_loop(0, NPW // L, mkdinv, 0)

    def prescale(k, _):
        pltpu.sync_copy(xw_hbm.at[pl.ds(row0g + k * RB, RB)], bufA)
        def prow(r, _):
            val = _splat(dinv_v, k * RB + r)
            for j in range(D // L):
                ebuf[r, pl.ds(j * L, L)] = ebuf[r, pl.ds(j * L, L)] * val
            return 0
        lax.fori_loop(0, RB, prow, 0)
        cpy = pltpu.async_copy(bufA, y_hbm.at[pl.ds(row0g + k * RB, RB)], sem)
        cpa = pltpu.async_copy(bufA, acc_sh.at[pl.ds(row0l + k * RB, RB)],
                               sem2)
        cpy.wait()
        cpa.wait()
        return 0
    lax.fori_loop(0, NB, prescale, 0)
    plsc.subcore_barrier()

    def edge_blk(blk, _):
        pltpu.sync_copy(srcg_hbm.at[c, s, pl.ds(blk * SG, SG)], sidx)
        pltpu.sync_copy(dstl_hbm.at[c, s, pl.ds(blk * SG, SG)], didx)

        pltpu.async_copy(y_hbm.at[sidx.at[0]], bufA, sem).wait()
        g1 = pltpu.async_copy(y_hbm.at[sidx.at[1]], bufB, sem)
        s0 = pltpu.async_copy(bufA, acc_sh.at[didx.at[0]], sem2, add=True)
        g1.wait()
        s0.wait()

        def pair(q, _):
            gA = pltpu.async_copy(y_hbm.at[sidx.at[2 * q]], bufA, sem)
            sB = pltpu.async_copy(bufB, acc_sh.at[didx.at[2 * q - 1]],
                                  sem2, add=True)
            gA.wait()
            sB.wait()
            gB = pltpu.async_copy(y_hbm.at[sidx.at[2 * q + 1]], bufB, sem)
            sA = pltpu.async_copy(bufA, acc_sh.at[didx.at[2 * q]],
                                  sem2, add=True)
            gB.wait()
            sA.wait()
            return 0
        lax.fori_loop(1, SG // 2, pair, 0)

        pltpu.async_copy(bufB, acc_sh.at[didx.at[SG - 1]],
                         sem2, add=True).wait()
        return 0
    lax.fori_loop(0, NBLK, edge_blk, 0)
    plsc.subcore_barrier()

    def pool_chunk(k, _):
        pltpu.sync_copy(acc_sh.at[pl.ds(row0l + k * RB, RB)], bufA)
        def prow(r, _):
            val = _splat(dinv_v, k * RB + r)
            for j in range(D // L):
                ebuf[r, pl.ds(j * L, L)] = jnp.maximum(
                    ebuf[r, pl.ds(j * L, L)] * val
                    + bias_v[pl.ds(j * L, L)], 0.0)
            return 0
        lax.fori_loop(0, RB, prow, 0)
        pltpu.sync_copy(bufA, pooled_sh.at[batch_v.at[k]], add=True)
        return 0
    lax.fori_loop(0, NB, pool_chunk, 0)
    plsc.subcore_barrier()

    pltpu.sync_copy(cnt_sh, cntv_v)
    pltpu.sync_copy(pooled_sh.at[pl.ds(g0, GPW)], ptmp)

    def divrow(r, _):
        cvec = _splat(cntv_v, g0 + r)
        recip = 1.0 / jnp.maximum(cvec, 1.0)
        for j in range(D // L):
            ptmp[r, pl.ds(j * L, L)] = ptmp[r, pl.ds(j * L, L)] * recip
        return 0
    lax.fori_loop(0, GPW, divrow, 0)

    pltpu.sync_copy(ptmp, out_hbm.at[c, pl.ds(g0, GPW)])


def _sc_call(xw, srcg, dstl, batchp, bias):
    mesh = plsc.VectorSubcoreMesh(core_axis_name="c", subcore_axis_name="s",
                                  num_cores=NC, num_subcores=NS)
    f = pl.kernel(
        _sc_body,
        out_type=(jax.ShapeDtypeStruct((NC * N_PAD, D), jnp.float32),
                  jax.ShapeDtypeStruct((NC, G, D), jnp.float32)),
        mesh=mesh,
        compiler_params=pltpu.CompilerParams(needs_layout_passes=False),
        scratch_types=[
            pltpu.VMEM((SG, RB), jnp.int32),
            pltpu.VMEM((SG, RB), jnp.int32),
            pltpu.VMEM((NPW,), jnp.float32),
            pltpu.VMEM((NPW,), jnp.float32),
            pltpu.VMEM((2 * RB, D), jnp.float32),
            pltpu.VMEM((GPP,), jnp.float32),
            pltpu.VMEM((NB, RB), jnp.int32),
            pltpu.VMEM((D,), jnp.float32),
            pltpu.VMEM((RB,), jnp.float32),
            pltpu.VMEM((GPW, D), jnp.float32),
            pltpu.VMEM_SHARED((N_PAD,), jnp.float32),
            pltpu.VMEM_SHARED((N_PAD, D), jnp.float32),
            pltpu.VMEM_SHARED((GP, D), jnp.float32),
            pltpu.VMEM_SHARED((GPP,), jnp.float32),
            pltpu.SemaphoreType.DMA,
            pltpu.SemaphoreType.DMA,
        ],
    )
    return f(xw, srcg, dstl, batchp, bias)


def kernel(x1, edge_index1, batch1, x2, edge_index2, batch2,
           W_gcn, b_gcn, W_link, b_link):
    f32 = jnp.float32
    i32 = jnp.int32
    e1 = edge_index1.astype(i32)
    e2 = edge_index2.astype(i32)

    xp = jnp.concatenate([
        jnp.pad(x1.astype(f32), ((0, N_PAD - N_NODES), (0, 0))),
        jnp.pad(x2.astype(f32), ((0, N_PAD - N_NODES), (0, 0))),
    ], axis=0)
    xw = _matmul(xp, W_gcn.astype(f32))

    pad_e = E_PAD - e1.shape[1]

    def prep(e, off):
        src = jnp.concatenate(
            [e[0] + i32(off), jnp.full((pad_e,), off + N_PAD - 1, i32)])
        dst = jnp.concatenate(
            [e[1], jnp.full((pad_e,), N_PAD - 1, i32)])
        return src.reshape(NS, NCHUNK, RB), dst.reshape(NS, NCHUNK, RB)

    s1, d1 = prep(e1, 0)
    s2, d2 = prep(e2, N_PAD)
    srcg = jnp.stack([s1, s2])
    dstl = jnp.stack([d1, d2])
    batchp = jnp.stack([
        jnp.pad(batch1.astype(i32), (0, N_PAD - N_NODES), constant_values=G),
        jnp.pad(batch2.astype(i32), (0, N_PAD - N_NODES), constant_values=G),
    ]).reshape(NC, NS, NB, RB)

    _y, pooled = _sc_call(xw, srcg, dstl, batchp, b_gcn.astype(f32))
    return _linear(pooled, W_link.astype(f32),
                   b_link.reshape(1, CLS).astype(f32))
